# baseline (speedup 1.0000x reference)
import functools

import jax
import jax.numpy as jnp
from jax import lax
from jax.experimental import pallas as pl
from jax.experimental.pallas import tpu as pltpu
from jax.experimental.pallas import tpu_sc as plsc

_NUM_CORES = 1
_NUM_SUBCORES = 8
_LANES = 16
_NUM_WORKERS = _NUM_CORES * _NUM_SUBCORES


def _make_gather(B, V, D):
    b_per_w = B // _NUM_WORKERS
    mesh = plsc.VectorSubcoreMesh(
        core_axis_name="c",
        subcore_axis_name="s",
        num_cores=_NUM_CORES,
        num_subcores=_NUM_SUBCORES,
    )

    @functools.partial(
        pl.kernel,
        mesh=mesh,
        out_type=jax.ShapeDtypeStruct((B, D), jnp.float32),
        scratch_types=[
            pltpu.VMEM((b_per_w,), jnp.int32),
            pltpu.VMEM((b_per_w, D), jnp.float32),
            pltpu.SemaphoreType.DMA,
            pltpu.SemaphoreType.DMA,
            pltpu.SemaphoreType.DMA,
        ],
    )
    def gather_kernel(table_hbm, idx_hbm, out_hbm, idx_v, rows_v, g0s, g1s, sts):
        wid = lax.axis_index("s") * _NUM_CORES + lax.axis_index("c")
        base = wid * b_per_w
        pltpu.sync_copy(idx_hbm.at[pl.ds(base, b_per_w)], idx_v)
        for t in range(b_per_w // _LANES):
            seg = idx_v[pl.ds(t * _LANES, _LANES)]
            row = lax.iota(jnp.int32, _LANES) + (base + t * _LANES)
            idx_v[pl.ds(t * _LANES, _LANES)] = seg + row * V
        h = b_per_w // 2
        g0 = pltpu.async_copy(
            table_hbm.at[idx_v.at[pl.ds(0, h)]], rows_v.at[pl.ds(0, h)], g0s
        )
        g1 = pltpu.async_copy(
            table_hbm.at[idx_v.at[pl.ds(h, h)]], rows_v.at[pl.ds(h, h)], g1s
        )
        g0.wait()
        s0 = pltpu.async_copy(
            rows_v.at[pl.ds(0, h)], out_hbm.at[pl.ds(base, h)], sts
        )
        g1.wait()
        pltpu.sync_copy(rows_v.at[pl.ds(h, h)], out_hbm.at[pl.ds(base + h, h)])
        s0.wait()

    return gather_kernel


def kernel(embeddings, current_node):
    B, V, D = embeddings.shape
    table = embeddings.reshape(B * V, D)
    idx = current_node.reshape(B)
    out = _make_gather(B, V, D)(table, idx)
    return out.reshape(B, 1, D)


# speedup vs baseline: 1.0295x; 1.0295x over previous
import functools

import jax
import jax.numpy as jnp
from jax import lax
from jax.experimental import pallas as pl
from jax.experimental.pallas import tpu as pltpu
from jax.experimental.pallas import tpu_sc as plsc

_NUM_CORES = 1
_NUM_SUBCORES = 16
_LANES = 16
_NUM_WORKERS = _NUM_CORES * _NUM_SUBCORES


def _make_gather(B, V, D):
    b_per_w = B // _NUM_WORKERS
    mesh = plsc.VectorSubcoreMesh(
        core_axis_name="c",
        subcore_axis_name="s",
        num_cores=_NUM_CORES,
        num_subcores=_NUM_SUBCORES,
    )

    @functools.partial(
        pl.kernel,
        mesh=mesh,
        out_type=jax.ShapeDtypeStruct((B, D), jnp.float32),
        scratch_types=[
            pltpu.VMEM((b_per_w,), jnp.int32),
            pltpu.VMEM((b_per_w, D), jnp.float32),
            pltpu.SemaphoreType.DMA,
            pltpu.SemaphoreType.DMA,
            pltpu.SemaphoreType.DMA,
        ],
    )
    def gather_kernel(table_hbm, idx_hbm, out_hbm, idx_v, rows_v, g0s, g1s, sts):
        wid = lax.axis_index("s") * _NUM_CORES + lax.axis_index("c")
        base = wid * b_per_w
        pltpu.sync_copy(idx_hbm.at[pl.ds(base, b_per_w)], idx_v)
        for t in range(b_per_w // _LANES):
            seg = idx_v[pl.ds(t * _LANES, _LANES)]
            row = lax.iota(jnp.int32, _LANES) + (base + t * _LANES)
            idx_v[pl.ds(t * _LANES, _LANES)] = seg + row * V
        h = b_per_w // 2
        g0 = pltpu.async_copy(
            table_hbm.at[idx_v.at[pl.ds(0, h)]], rows_v.at[pl.ds(0, h)], g0s
        )
        g1 = pltpu.async_copy(
            table_hbm.at[idx_v.at[pl.ds(h, h)]], rows_v.at[pl.ds(h, h)], g1s
        )
        g0.wait()
        s0 = pltpu.async_copy(
            rows_v.at[pl.ds(0, h)], out_hbm.at[pl.ds(base, h)], sts
        )
        g1.wait()
        pltpu.sync_copy(rows_v.at[pl.ds(h, h)], out_hbm.at[pl.ds(base + h, h)])
        s0.wait()

    return gather_kernel


def kernel(embeddings, current_node):
    B, V, D = embeddings.shape
    table = embeddings.reshape(B * V, D)
    idx = current_node.reshape(B)
    out = _make_gather(B, V, D)(table, idx)
    return out.reshape(B, 1, D)
